# baseline (speedup 1.0000x reference)
import functools

import jax
import jax.numpy as jnp
from jax import lax
from jax.experimental import pallas as pl
from jax.experimental.pallas import tpu as pltpu
from jax.experimental.pallas import tpu_sc as plsc

B, NU, N, K = 2, 4096, 16384, 16
DU, DP, H, DO = 256, 256, 64, 256
H2 = 2 * H
BN = B * N
BNU = B * NU

NC, NS = 2, 16
NW = NC * NS
PTSW = BN // NW
BLK = 16
NBLK = PTSW // BLK


def _lrelu(v):
    return jnp.where(v >= 0, v, 0.1 * v)


def _mlp(x, W1, b1, W2, b2, blk_rows, pad128=False, concat_right=None):
    R, Fin = x.shape
    Hh = W1.shape[1]
    wide = pad128 or concat_right is not None
    wout = H2 if wide else Hh

    def body(*refs):
        if concat_right is None:
            x_ref, w1_ref, b1_ref, w2_ref, b2_ref, o_ref = refs
        else:
            x_ref, cr_ref, w1_ref, b1_ref, w2_ref, b2_ref, o_ref = refs
        h = jnp.dot(x_ref[...], w1_ref[...], preferred_element_type=jnp.float32)
        h = _lrelu(h + b1_ref[...])
        o = jnp.dot(h, w2_ref[...], preferred_element_type=jnp.float32)
        o = o + b2_ref[...]
        if pad128:
            o = jnp.concatenate([jnp.zeros_like(o), o], axis=1)
        elif concat_right is not None:
            o = jnp.concatenate([o, cr_ref[:, Hh:]], axis=1)
        o_ref[...] = o

    in_specs = [pl.BlockSpec((blk_rows, Fin), lambda i: (i, 0))]
    args = [x]
    if concat_right is not None:
        in_specs.append(pl.BlockSpec((blk_rows, H2), lambda i: (i, 0)))
        args.append(concat_right)
    in_specs += [
        pl.BlockSpec((Fin, Hh), lambda i: (0, 0)),
        pl.BlockSpec((1, Hh), lambda i: (0, 0)),
        pl.BlockSpec((Hh, Hh), lambda i: (0, 0)),
        pl.BlockSpec((1, Hh), lambda i: (0, 0)),
    ]
    args += [W1, b1.reshape(1, Hh), W2, b2.reshape(1, Hh)]

    return pl.pallas_call(
        body,
        grid=(R // blk_rows,),
        in_specs=in_specs,
        out_specs=pl.BlockSpec((blk_rows, wout), lambda i: (i, 0)),
        out_shape=jax.ShapeDtypeStruct((R, wout), jnp.float32),
    )(*args)


def _sc_gather_rows(table, gidx):
    rows_total = gidx.shape[0]
    rows_per_w = rows_total // NW
    CH = 512
    nch = rows_per_w // CH
    mesh = plsc.VectorSubcoreMesh(core_axis_name="c", subcore_axis_name="s")

    @functools.partial(
        pl.kernel,
        out_type=jax.ShapeDtypeStruct((rows_total, H2), jnp.float32),
        mesh=mesh,
        compiler_params=pltpu.CompilerParams(needs_layout_passes=False),
        scratch_types=[
            pltpu.VMEM((CH,), jnp.int32),
            pltpu.VMEM((CH, H2), jnp.float32),
            pltpu.SemaphoreType.DMA,
        ],
    )
    def gk(table_hbm, idx_hbm, out_hbm, idx_v, rows_v, sem):
        wid = lax.axis_index("s") * NC + lax.axis_index("c")
        base = wid * rows_per_w

        def chunk(g, carry):
            r0 = pl.multiple_of(base + g * CH, 8)
            pltpu.sync_copy(idx_hbm.at[pl.ds(r0, CH)], idx_v)
            pltpu.async_copy(table_hbm.at[idx_v], rows_v, sem).wait()
            pltpu.sync_copy(rows_v, out_hbm.at[pl.ds(r0, CH)])
            return carry

        lax.fori_loop(0, nch, chunk, 0)

    return gk(table, gidx)


def _sc_crf_aggregate(tab, gidx):
    mesh = plsc.VectorSubcoreMesh(core_axis_name="c", subcore_axis_name="s")

    @functools.partial(
        pl.kernel,
        out_type=jax.ShapeDtypeStruct((BN, H), jnp.float32),
        mesh=mesh,
        compiler_params=pltpu.CompilerParams(needs_layout_passes=False),
        scratch_types=[
            pltpu.VMEM((BLK * (K + 1),), jnp.int32),
            pltpu.VMEM((BLK * (K + 1),), jnp.int32),
            pltpu.VMEM((BLK * (K + 1), H2), jnp.float32),
            pltpu.VMEM((BLK * (K + 1), H2), jnp.float32),
            pltpu.VMEM((BLK, H), jnp.float32),
            pltpu.VMEM((BLK, H), jnp.float32),
            pltpu.SemaphoreType.DMA,
            pltpu.SemaphoreType.DMA,
            pltpu.SemaphoreType.DMA,
            pltpu.SemaphoreType.DMA,
            pltpu.SemaphoreType.DMA,
            pltpu.SemaphoreType.DMA,
        ],
    )
    def ck(tab_hbm, gidx_hbm, out_hbm, idx_v0, idx_v1, nb_v0, nb_v1,
           o_v0, o_v1, sem_nb0, sem_nb1, sem_nc0, sem_nc1, sem_o0, sem_o1):
        idx_vs = (idx_v0, idx_v1)
        nb_vs = (nb_v0, nb_v1)
        o_vs = (o_v0, o_v1)
        sem_nb = (sem_nb0, sem_nb1)
        sem_nc = (sem_nc0, sem_nc1)
        sem_o = (sem_o0, sem_o1)
        HALF = BLK * (K + 1) // 2
        wid = lax.axis_index("s") * NC + lax.axis_index("c")
        base = wid * PTSW
        lane = lax.iota(jnp.int32, 16)
        lmask = [lane == k for k in range(K)]

        def r0_of(g):
            return pl.multiple_of(base + g * BLK, 8)

        def issue(g, s):
            i0 = pl.multiple_of((base + g * BLK) * K, 8)
            pltpu.sync_copy(gidx_hbm.at[pl.ds(i0, BLK * K)],
                            idx_vs[s].at[pl.ds(0, BLK * K)])
            idx_vs[s][pl.ds(BLK * K, 16)] = lane + r0_of(g)
            pltpu.async_copy(tab_hbm.at[idx_vs[s]], nb_vs[s], sem_nb[s])

        def wait_in(g, s):
            pltpu.make_async_copy(tab_hbm.at[idx_vs[s]], nb_vs[s],
                                  sem_nb[s]).wait()

        NCH = H // 16

        def compute(g, s):
            nbr = nb_vs[s]
            orr = o_vs[s]

            def point_body(p):
                yc = [nbr[BLK * K + p, pl.ds(c * 16, 16)]
                      for c in range(NCH)]
                dk = []
                for k in range(K):
                    r = p * K + k
                    d = nbr[r, pl.ds(0, 16)] - yc[0]
                    acc = d * d
                    for c in range(1, NCH):
                        d = nbr[r, pl.ds(c * 16, 16)] - yc[c]
                        acc = acc + d * d
                    dk.append(jnp.sum(acc))

                dvec = jnp.full((16,), dk[0], jnp.float32)
                for k in range(1, K):
                    dvec = jnp.where(lmask[k],
                                     jnp.full((16,), dk[k], jnp.float32),
                                     dvec)
                m = jnp.min(dvec)
                e = jnp.exp(m - dvec)
                srow = e / jnp.sum(e)
                sv = [srow[k] for k in range(K)]

                for c in range(NCH):
                    acc = sv[0] * nbr[p * K, pl.ds(H + c * 16, 16)]
                    for k in range(1, K):
                        acc = acc + sv[k] * nbr[p * K + k,
                                                pl.ds(H + c * 16, 16)]
                    orr[p, pl.ds(c * 16, 16)] = acc

            plsc.parallel_loop(0, BLK, unroll=1)(point_body)

            pltpu.async_copy(orr, out_hbm.at[pl.ds(r0_of(g), BLK)], sem_o[s])

        issue(0, 0)
        issue(1, 1)

        def outer(gg, carry):
            g0 = gg * 2
            for b in range(2):
                g = g0 + b

                @pl.when(g >= 2)
                def _():
                    pltpu.make_async_copy(
                        o_vs[b], out_hbm.at[pl.ds(r0_of(g - 2), BLK)],
                        sem_o[b]).wait()

                wait_in(g, b)
                compute(g, b)

                @pl.when(g + 2 < NBLK)
                def _():
                    issue(g + 2, b)
            return carry

        lax.fori_loop(0, NBLK // 2, outer, 0)
        for b in range(2):
            pltpu.make_async_copy(
                o_vs[b], out_hbm.at[pl.ds(r0_of(NBLK - 2 + b), BLK)],
                sem_o[b]).wait()

    return ck(tab, gidx)


def _tail(agg128, t128, pw, Cm, Cinv, W_o, b_o, Wf1, Wf2, b_f, blk_rows):
    R = agg128.shape[0]

    def body(agg_ref, t_ref, pw_ref, c_ref, ci_ref, wo_ref, bo_ref,
             wf1_ref, wf2_ref, bf_ref, o_ref):
        agg = agg_ref[...]
        z = t_ref[:, H:H2]
        x = z + jnp.dot(agg, c_ref[...], preferred_element_type=jnp.float32)
        x = jnp.dot(x, ci_ref[...], preferred_element_type=jnp.float32)
        o = jnp.dot(x.astype(jnp.bfloat16), wo_ref[...].astype(jnp.bfloat16),
                    preferred_element_type=jnp.float32)
        o = _lrelu(o + bo_ref[...])
        t = jnp.dot(o.astype(jnp.bfloat16), wf1_ref[...].astype(jnp.bfloat16),
                    preferred_element_type=jnp.float32)
        t = t + jnp.dot(pw_ref[...].astype(jnp.bfloat16),
                        wf2_ref[...].astype(jnp.bfloat16),
                        preferred_element_type=jnp.float32)
        o_ref[...] = _lrelu(t + bf_ref[...])

    return pl.pallas_call(
        body,
        grid=(R // blk_rows,),
        in_specs=[
            pl.BlockSpec((blk_rows, H), lambda i: (i, 0)),
            pl.BlockSpec((blk_rows, H2), lambda i: (i, 0)),
            pl.BlockSpec((blk_rows, DP), lambda i: (i, 0)),
            pl.BlockSpec((H, H), lambda i: (0, 0)),
            pl.BlockSpec((H, H), lambda i: (0, 0)),
            pl.BlockSpec((H, DO), lambda i: (0, 0)),
            pl.BlockSpec((1, DO), lambda i: (0, 0)),
            pl.BlockSpec((DO, DO), lambda i: (0, 0)),
            pl.BlockSpec((DP, DO), lambda i: (0, 0)),
            pl.BlockSpec((1, DO), lambda i: (0, 0)),
        ],
        out_specs=pl.BlockSpec((blk_rows, DO), lambda i: (i, 0)),
        out_shape=jax.ShapeDtypeStruct((R, DO), jnp.float32),
    )(agg128, t128, pw, Cm, Cinv, W_o, b_o.reshape(1, DO), Wf1, Wf2,
      b_f.reshape(1, DO))


def kernel(unary, pairwise, up_idx, neighbor_idx, W_u1, b_u1, W_u2, b_u2,
           W_p1, b_p1, W_p2, b_p2, W_o, b_o, W_f, b_f, c):
    boff_u = (jnp.arange(B, dtype=jnp.int32) * NU)[:, None]
    gup = (up_idx[..., 0].astype(jnp.int32) + boff_u).reshape(BN)
    nidx = neighbor_idx[:, :, 1:].astype(jnp.int32)
    boff_n = (jnp.arange(B, dtype=jnp.int32) * N)[:, None, None]
    gidx = (nidx + boff_n).reshape(BN * K)

    u2p = _mlp(unary.reshape(BNU, DU), W_u1, b_u1, W_u2, b_u2, 1024,
               pad128=True)

    xupP = _sc_gather_rows(u2p, gup)

    t128 = _mlp(pairwise.reshape(BN, DP), W_p1, b_p1, W_p2, b_p2, 2048,
                concat_right=xupP)

    agg = _sc_crf_aggregate(t128, gidx)

    Cm = c.T @ c
    Cinv = jnp.linalg.inv(jnp.eye(H, dtype=jnp.float32) + Cm)
    Wf1 = W_f[:DO]
    Wf2 = W_f[DO:]

    out = _tail(agg, t128, pairwise.reshape(BN, DP), Cm, Cinv,
                W_o, b_o, Wf1, Wf2, b_f, 2048)
    return out.reshape(B, N, DO)


# speedup vs baseline: 1.0597x; 1.0597x over previous
import functools

import jax
import jax.numpy as jnp
from jax import lax
from jax.experimental import pallas as pl
from jax.experimental.pallas import tpu as pltpu
from jax.experimental.pallas import tpu_sc as plsc

B, NU, N, K = 2, 4096, 16384, 16
DU, DP, H, DO = 256, 256, 64, 256
H2 = 2 * H
BN = B * N
BNU = B * NU

NC, NS = 2, 16
NW = NC * NS
PTSW = BN // NW
BLK = 16
NBLK = PTSW // BLK


def _lrelu(v):
    return jnp.where(v >= 0, v, 0.1 * v)


def _mlp(x, W1, b1, W2, b2, blk_rows, pad128=False, concat_right=None):
    R, Fin = x.shape
    Hh = W1.shape[1]
    wide = pad128 or concat_right is not None
    wout = H2 if wide else Hh

    def body(*refs):
        if concat_right is None:
            x_ref, w1_ref, b1_ref, w2_ref, b2_ref, o_ref = refs
        else:
            x_ref, cr_ref, w1_ref, b1_ref, w2_ref, b2_ref, o_ref = refs
        h = jnp.dot(x_ref[...], w1_ref[...], preferred_element_type=jnp.float32)
        h = _lrelu(h + b1_ref[...])
        o = jnp.dot(h, w2_ref[...], preferred_element_type=jnp.float32)
        o = o + b2_ref[...]
        if pad128:
            o = jnp.concatenate([jnp.zeros_like(o), o], axis=1)
        elif concat_right is not None:
            o = jnp.concatenate([o, cr_ref[:, Hh:]], axis=1)
        o_ref[...] = o

    in_specs = [pl.BlockSpec((blk_rows, Fin), lambda i: (i, 0))]
    args = [x]
    if concat_right is not None:
        in_specs.append(pl.BlockSpec((blk_rows, H2), lambda i: (i, 0)))
        args.append(concat_right)
    in_specs += [
        pl.BlockSpec((Fin, Hh), lambda i: (0, 0)),
        pl.BlockSpec((1, Hh), lambda i: (0, 0)),
        pl.BlockSpec((Hh, Hh), lambda i: (0, 0)),
        pl.BlockSpec((1, Hh), lambda i: (0, 0)),
    ]
    args += [W1, b1.reshape(1, Hh), W2, b2.reshape(1, Hh)]

    return pl.pallas_call(
        body,
        grid=(R // blk_rows,),
        in_specs=in_specs,
        out_specs=pl.BlockSpec((blk_rows, wout), lambda i: (i, 0)),
        out_shape=jax.ShapeDtypeStruct((R, wout), jnp.float32),
    )(*args)


def _sc_gather_rows(table, gidx):
    rows_total = gidx.shape[0]
    rows_per_w = rows_total // NW
    CH = 512
    nch = rows_per_w // CH
    mesh = plsc.VectorSubcoreMesh(core_axis_name="c", subcore_axis_name="s")

    @functools.partial(
        pl.kernel,
        out_type=jax.ShapeDtypeStruct((rows_total, H2), jnp.float32),
        mesh=mesh,
        compiler_params=pltpu.CompilerParams(needs_layout_passes=False),
        scratch_types=[
            pltpu.VMEM((CH,), jnp.int32),
            pltpu.VMEM((CH, H2), jnp.float32),
            pltpu.SemaphoreType.DMA,
        ],
    )
    def gk(table_hbm, idx_hbm, out_hbm, idx_v, rows_v, sem):
        wid = lax.axis_index("s") * NC + lax.axis_index("c")
        base = wid * rows_per_w

        def chunk(g, carry):
            r0 = pl.multiple_of(base + g * CH, 8)
            pltpu.sync_copy(idx_hbm.at[pl.ds(r0, CH)], idx_v)
            pltpu.async_copy(table_hbm.at[idx_v], rows_v, sem).wait()
            pltpu.sync_copy(rows_v, out_hbm.at[pl.ds(r0, CH)])
            return carry

        lax.fori_loop(0, nch, chunk, 0)

    return gk(table, gidx)


def _sc_crf_aggregate(tab, gidx):
    mesh = plsc.VectorSubcoreMesh(core_axis_name="c", subcore_axis_name="s")

    @functools.partial(
        pl.kernel,
        out_type=jax.ShapeDtypeStruct((BN, H), jnp.float32),
        mesh=mesh,
        compiler_params=pltpu.CompilerParams(needs_layout_passes=False),
        scratch_types=[
            pltpu.VMEM((BLK * (K + 1),), jnp.int32),
            pltpu.VMEM((BLK * (K + 1),), jnp.int32),
            pltpu.VMEM((BLK * (K + 1),), jnp.int32),
            pltpu.VMEM((BLK * (K + 1), H2), jnp.float32),
            pltpu.VMEM((BLK * (K + 1), H2), jnp.float32),
            pltpu.VMEM((BLK * (K + 1), H2), jnp.float32),
            pltpu.VMEM((BLK, H), jnp.float32),
            pltpu.VMEM((BLK, H), jnp.float32),
            pltpu.VMEM((BLK, H), jnp.float32),
            pltpu.SemaphoreType.DMA,
            pltpu.SemaphoreType.DMA,
            pltpu.SemaphoreType.DMA,
            pltpu.SemaphoreType.DMA,
            pltpu.SemaphoreType.DMA,
            pltpu.SemaphoreType.DMA,
        ],
    )
    def ck(tab_hbm, gidx_hbm, out_hbm, idx_v0, idx_v1, idx_v2,
           nb_v0, nb_v1, nb_v2, o_v0, o_v1, o_v2,
           sem_nb0, sem_nb1, sem_nb2, sem_o0, sem_o1, sem_o2):
        idx_vs = (idx_v0, idx_v1, idx_v2)
        nb_vs = (nb_v0, nb_v1, nb_v2)
        o_vs = (o_v0, o_v1, o_v2)
        sem_nb = (sem_nb0, sem_nb1, sem_nb2)
        sem_o = (sem_o0, sem_o1, sem_o2)
        wid = lax.axis_index("s") * NC + lax.axis_index("c")
        base = wid * PTSW
        lane = lax.iota(jnp.int32, 16)
        lmask = [lane == k for k in range(K)]

        def r0_of(g):
            return pl.multiple_of(base + g * BLK, 8)

        def issue(g, s):
            i0 = pl.multiple_of((base + g * BLK) * K, 8)
            pltpu.sync_copy(gidx_hbm.at[pl.ds(i0, BLK * K)],
                            idx_vs[s].at[pl.ds(0, BLK * K)])
            idx_vs[s][pl.ds(BLK * K, 16)] = lane + r0_of(g)
            pltpu.async_copy(tab_hbm.at[idx_vs[s]], nb_vs[s], sem_nb[s])

        def wait_in(g, s):
            pltpu.make_async_copy(tab_hbm.at[idx_vs[s]], nb_vs[s],
                                  sem_nb[s]).wait()

        NCH = H // 16

        def compute(g, s):
            nbr = nb_vs[s]
            orr = o_vs[s]

            def point_body(p):
                yc = [nbr[BLK * K + p, pl.ds(c * 16, 16)]
                      for c in range(NCH)]
                dk = []
                for k in range(K):
                    r = p * K + k
                    d = nbr[r, pl.ds(0, 16)] - yc[0]
                    acc = d * d
                    for c in range(1, NCH):
                        d = nbr[r, pl.ds(c * 16, 16)] - yc[c]
                        acc = acc + d * d
                    dk.append(jnp.sum(acc))

                dvec = jnp.full((16,), dk[0], jnp.float32)
                for k in range(1, K):
                    dvec = jnp.where(lmask[k],
                                     jnp.full((16,), dk[k], jnp.float32),
                                     dvec)
                m = jnp.min(dvec)
                e = jnp.exp(m - dvec)
                srow = e / jnp.sum(e)
                sv = [srow[k] for k in range(K)]

                for c in range(NCH):
                    acc = sv[0] * nbr[p * K, pl.ds(H + c * 16, 16)]
                    for k in range(1, K):
                        acc = acc + sv[k] * nbr[p * K + k,
                                                pl.ds(H + c * 16, 16)]
                    orr[p, pl.ds(c * 16, 16)] = acc

            plsc.parallel_loop(0, BLK, unroll=2)(point_body)

            pltpu.async_copy(orr, out_hbm.at[pl.ds(r0_of(g), BLK)], sem_o[s])

        issue(0, 0)
        issue(1, 1)
        issue(2, 2)

        def step(g, b):
            @pl.when(g >= 3)
            def _():
                pltpu.make_async_copy(
                    o_vs[b], out_hbm.at[pl.ds(r0_of(g - 3), BLK)],
                    sem_o[b]).wait()

            wait_in(g, b)
            compute(g, b)

            @pl.when(g + 3 < NBLK)
            def _():
                issue(g + 3, b)

        def outer(gg, carry):
            g0 = gg * 3
            for b in range(3):
                step(g0 + b, b)
            return carry

        lax.fori_loop(0, NBLK // 3, outer, 0)
        for g in range(NBLK - NBLK % 3, NBLK):
            step(g, g % 3)
        for g in range(NBLK - 3, NBLK):
            pltpu.make_async_copy(
                o_vs[g % 3], out_hbm.at[pl.ds(r0_of(g), BLK)],
                sem_o[g % 3]).wait()

    return ck(tab, gidx)


def _tail(agg128, t128, pw, Cm, Cinv, W_o, b_o, Wf1, Wf2, b_f, blk_rows):
    R = agg128.shape[0]

    def body(agg_ref, t_ref, pw_ref, c_ref, ci_ref, wo_ref, bo_ref,
             wf1_ref, wf2_ref, bf_ref, o_ref):
        agg = agg_ref[...]
        z = t_ref[:, H:H2]
        x = z + jnp.dot(agg, c_ref[...], preferred_element_type=jnp.float32)
        x = jnp.dot(x, ci_ref[...], preferred_element_type=jnp.float32)
        o = jnp.dot(x.astype(jnp.bfloat16), wo_ref[...].astype(jnp.bfloat16),
                    preferred_element_type=jnp.float32)
        o = _lrelu(o + bo_ref[...])
        t = jnp.dot(o.astype(jnp.bfloat16), wf1_ref[...].astype(jnp.bfloat16),
                    preferred_element_type=jnp.float32)
        t = t + jnp.dot(pw_ref[...].astype(jnp.bfloat16),
                        wf2_ref[...].astype(jnp.bfloat16),
                        preferred_element_type=jnp.float32)
        o_ref[...] = _lrelu(t + bf_ref[...])

    return pl.pallas_call(
        body,
        grid=(R // blk_rows,),
        in_specs=[
            pl.BlockSpec((blk_rows, H), lambda i: (i, 0)),
            pl.BlockSpec((blk_rows, H2), lambda i: (i, 0)),
            pl.BlockSpec((blk_rows, DP), lambda i: (i, 0)),
            pl.BlockSpec((H, H), lambda i: (0, 0)),
            pl.BlockSpec((H, H), lambda i: (0, 0)),
            pl.BlockSpec((H, DO), lambda i: (0, 0)),
            pl.BlockSpec((1, DO), lambda i: (0, 0)),
            pl.BlockSpec((DO, DO), lambda i: (0, 0)),
            pl.BlockSpec((DP, DO), lambda i: (0, 0)),
            pl.BlockSpec((1, DO), lambda i: (0, 0)),
        ],
        out_specs=pl.BlockSpec((blk_rows, DO), lambda i: (i, 0)),
        out_shape=jax.ShapeDtypeStruct((R, DO), jnp.float32),
    )(agg128, t128, pw, Cm, Cinv, W_o, b_o.reshape(1, DO), Wf1, Wf2,
      b_f.reshape(1, DO))


def kernel(unary, pairwise, up_idx, neighbor_idx, W_u1, b_u1, W_u2, b_u2,
           W_p1, b_p1, W_p2, b_p2, W_o, b_o, W_f, b_f, c):
    boff_u = (jnp.arange(B, dtype=jnp.int32) * NU)[:, None]
    gup = (up_idx[..., 0].astype(jnp.int32) + boff_u).reshape(BN)
    nidx = neighbor_idx[:, :, 1:].astype(jnp.int32)
    boff_n = (jnp.arange(B, dtype=jnp.int32) * N)[:, None, None]
    gidx = (nidx + boff_n).reshape(BN * K)

    u2p = _mlp(unary.reshape(BNU, DU), W_u1, b_u1, W_u2, b_u2, 1024,
               pad128=True)

    xupP = _sc_gather_rows(u2p, gup)

    t128 = _mlp(pairwise.reshape(BN, DP), W_p1, b_p1, W_p2, b_p2, 2048,
                concat_right=xupP)

    agg = _sc_crf_aggregate(t128, gidx)

    Cm = c.T @ c
    Cinv = jnp.linalg.inv(jnp.eye(H, dtype=jnp.float32) + Cm)
    Wf1 = W_f[:DO]
    Wf2 = W_f[DO:]

    out = _tail(agg, t128, pairwise.reshape(BN, DP), Cm, Cinv,
                W_o, b_o, Wf1, Wf2, b_f, 2048)
    return out.reshape(B, N, DO)
